# Initial kernel scaffold; baseline (speedup 1.0000x reference)
import jax
import jax.numpy as jnp
from jax.experimental import pallas as pl


def kernel(x, target_sequence):
    raise NotImplementedError("write your pallas kernel here")


# speedup vs baseline: 7.4730x; 7.4730x over previous
import math

import jax
import jax.numpy as jnp
from jax.experimental import pallas as pl
from jax.experimental.pallas import tpu as pltpu

SMOOTH = 0.1
CONF = 1.0 - SMOOTH
SEQ = 2048
VOCAB = 32000
TILE = 3200
NT = VOCAB // TILE
EPS = SMOOTH / (VOCAB - 2)
KCONST = CONF * math.log(CONF) + (VOCAB - 2) * EPS * math.log(EPS)


def _body(t_ref, x_ref, out_ref):
    j = pl.program_id(0)
    t = t_ref[:, :1]
    m = (t != 0).astype(jnp.float32)
    x = x_ref[...]
    col = jax.lax.broadcasted_iota(jnp.int32, (SEQ, TILE), 1) + j * TILE
    w = jnp.where(col == t, EPS - CONF, -EPS) + jnp.where(col == 0, EPS, 0.0)
    contrib = jnp.sum(x * (w * m))

    @pl.when(j == 0)
    def _():
        out_ref[0, 0] = KCONST * jnp.sum(m)

    out_ref[0, 0] += contrib


def kernel(x, target_sequence):
    x2 = x.reshape(SEQ, VOCAB)
    t2 = target_sequence.reshape(SEQ, 1).astype(jnp.int32)
    out = pl.pallas_call(
        _body,
        grid=(NT,),
        in_specs=[
            pl.BlockSpec((SEQ, 1), lambda j: (0, 0)),
            pl.BlockSpec((SEQ, TILE), lambda j: (0, j)),
        ],
        out_specs=pl.BlockSpec(memory_space=pltpu.SMEM),
        out_shape=jax.ShapeDtypeStruct((1, 1), jnp.float32),
    )(t2, x2)
    return out[0, 0]
